# baseline (speedup 1.0000x reference)
import functools

import jax
import jax.numpy as jnp
from jax import lax
from jax.experimental import pallas as pl
from jax.experimental.pallas import tpu as pltpu
from jax.experimental.pallas import tpu_sc as plsc

EF = 128
KNN = 8

_NC = 2
_NS = 16
_NW = _NC * _NS

_CH = 448
_CPW0 = 35
_CPW1 = 21
_DEPTH = 2


def _h_matmul_body(x_ref, w_ref, o_ref):
    o_ref[...] = jnp.dot(x_ref[...], w_ref[...],
                         preferred_element_type=jnp.float32)


def _h_matmul(x, w_t):
    n = x.shape[0]
    bm = 5000
    grid = n // bm
    return pl.pallas_call(
        _h_matmul_body,
        grid=(grid,),
        in_specs=[
            pl.BlockSpec((bm, EF), lambda i: (i, 0)),
            pl.BlockSpec((EF, EF), lambda i: (0, 0)),
        ],
        out_specs=pl.BlockSpec((bm, EF), lambda i: (i, 0)),
        out_shape=jax.ShapeDtypeStruct((n, EF), jnp.float32),
    )(x, w_t)


def _make_gather_body(cpw0, cpw1):
    D = _DEPTH

    def body(h_hbm, idx_hbm, out_hbm, *bufs):
        idx_v = list(bufs[:D])
        rows_v, gsem, ssem = bufs[D], bufs[D + 1], bufs[D + 2]
        c = lax.axis_index("c")
        s = lax.axis_index("s")
        cpw = jnp.where(c == 0, cpw0, cpw1)
        base = jnp.where(c == 0, s * (_CH * cpw0),
                         _NS * (_CH * cpw0) + s * (_CH * cpw1))

        def wait_g(b):
            pltpu.make_async_copy(h_hbm.at[pl.ds(0, _CH)], rows_v.at[b],
                                  gsem.at[b]).wait()

        def wait_s(b):
            pltpu.make_async_copy(rows_v.at[b], out_hbm.at[pl.ds(0, _CH)],
                                  ssem.at[b]).wait()

        def start(i):
            b = i % D

            @pl.when(i < cpw)
            def _():
                pltpu.sync_copy(idx_hbm.at[pl.ds(base + i * _CH, _CH)],
                                idx_v[b])
                pltpu.async_copy(h_hbm.at[idx_v[b]], rows_v.at[b],
                                 gsem.at[b])

        for j in range(D - 1):
            start(j)
        for i in range(cpw0):
            b = i % D
            j = i + D - 1
            if j < cpw0:
                if j >= D:
                    @pl.when(j < cpw)
                    def _(j=j):
                        wait_s(j % D)
                start(j)

            @pl.when(i < cpw)
            def _(b=b, i=i):
                wait_g(b)
                pltpu.async_copy(rows_v.at[b],
                                 out_hbm.at[pl.ds(base + i * _CH, _CH)],
                                 ssem.at[b])
        for b in range(D):
            wait_s(b)

    return body


def _sc_gather(h, idx_pad, cpw0, cpw1):
    e_slice = _NS * (cpw0 + cpw1) * _CH
    mesh = plsc.VectorSubcoreMesh(core_axis_name="c", subcore_axis_name="s")
    k = pl.kernel(
        _make_gather_body(cpw0, cpw1),
        out_type=jax.ShapeDtypeStruct((e_slice, EF), jnp.float32),
        mesh=mesh,
        scratch_types=(
            [pltpu.VMEM((_CH,), jnp.int32)] * _DEPTH
            + [pltpu.VMEM((_DEPTH, _CH, EF), jnp.float32),
               pltpu.SemaphoreType.DMA((_DEPTH,)),
               pltpu.SemaphoreType.DMA((_DEPTH,))]
        ),
    )
    return k(h, idx_pad)


def _mlp_max_body(g_ref, xyz_ref, w1x_ref, b1_ref, w2t_ref, b2_ref, o_ref):
    xyzc = lax.dot_general(xyz_ref[0], w1x_ref[...],
                           dimension_numbers=(((0,), (0,)), ((), ())),
                           preferred_element_type=jnp.float32)
    pre = g_ref[...] + b1_ref[...] + xyzc
    act = jnp.where(pre >= 0, pre, 0.01 * pre)
    o2 = jnp.dot(act.astype(jnp.bfloat16), w2t_ref[...],
                 preferred_element_type=jnp.float32)
    bm = o2.shape[0]
    o_ref[...] = jnp.max(o2.reshape(bm // KNN, KNN, EF), axis=1) + b2_ref[...]


def _mlp_max(g, xyz_t, w1x_t, b1, w2_t, b2, n_nodes, bm):
    e = n_nodes * KNN
    grid = e // bm
    return pl.pallas_call(
        _mlp_max_body,
        grid=(grid,),
        in_specs=[
            pl.BlockSpec((bm, EF), lambda i: (i, 0)),
            pl.BlockSpec((1, 3, bm), lambda i: (i, 0, 0)),
            pl.BlockSpec((3, EF), lambda i: (0, 0)),
            pl.BlockSpec((1, EF), lambda i: (0, 0)),
            pl.BlockSpec((EF, EF), lambda i: (0, 0)),
            pl.BlockSpec((1, EF), lambda i: (0, 0)),
        ],
        out_specs=pl.BlockSpec((bm // KNN, EF), lambda i: (i, 0)),
        out_shape=jax.ShapeDtypeStruct((n_nodes, EF), jnp.float32),
    )(g, xyz_t, w1x_t, b1, w2_t, b2)


def kernel(input, KNN_idx, KNN_xyz, W1, b1, W2, b2):
    n = input.shape[0]
    e = KNN_idx.shape[0]

    idx = KNN_idx.astype(jnp.int32)
    e_pad = _NS * (_CPW0 + _CPW1) * _CH
    idx_pad = jnp.concatenate(
        [idx, jnp.zeros((e_pad - e,), dtype=jnp.int32)])

    w1f_t = W1[:, :EF].T
    w1x_t = W1[:, EF:].T
    w2_t = W2.T.astype(jnp.bfloat16)

    h = _h_matmul(input, w1f_t)
    g = _sc_gather(h, idx_pad, _CPW0, _CPW1)

    bm = 8000
    xyz_t = KNN_xyz.reshape(e // bm, bm, 3).transpose(0, 2, 1)

    return _mlp_max(g, xyz_t, w1x_t, b1.reshape(1, EF), w2_t,
                    b2.reshape(1, EF), n, bm)


# speedup vs baseline: 1.0011x; 1.0011x over previous
import jax
import jax.numpy as jnp
from jax import lax
from jax.experimental import pallas as pl
from jax.experimental.pallas import tpu as pltpu
from jax.experimental.pallas import tpu_sc as plsc

EF = 128
KNN = 8

_NC = 2
_NS = 16
_NW = _NC * _NS

_CH = 448
_CPW0 = 33
_CPW1 = 23
_DEPTH = 2


def _h_matmul_body(x_ref, w_ref, o_ref):
    o_ref[...] = jnp.dot(x_ref[...], w_ref[...],
                         preferred_element_type=jnp.float32)


def _h_matmul(x, w_t):
    n = x.shape[0]
    bm = 5000
    grid = n // bm
    return pl.pallas_call(
        _h_matmul_body,
        grid=(grid,),
        in_specs=[
            pl.BlockSpec((bm, EF), lambda i: (i, 0)),
            pl.BlockSpec((EF, EF), lambda i: (0, 0)),
        ],
        out_specs=pl.BlockSpec((bm, EF), lambda i: (i, 0)),
        out_shape=jax.ShapeDtypeStruct((n, EF), jnp.float32),
    )(x, w_t)


def _make_gather_body(cpw0, cpw1):
    D = _DEPTH

    def body(h_hbm, idx_hbm, out_hbm, *bufs):
        idx_v = list(bufs[:D])
        rows_v, gsem, ssem = bufs[D], bufs[D + 1], bufs[D + 2]
        c = lax.axis_index("c")
        s = lax.axis_index("s")
        cpw = jnp.where(c == 0, cpw0, cpw1)
        base = jnp.where(c == 0, s * (_CH * cpw0),
                         _NS * (_CH * cpw0) + s * (_CH * cpw1))

        def wait_g(b):
            pltpu.make_async_copy(h_hbm.at[pl.ds(0, _CH)], rows_v.at[b],
                                  gsem.at[b]).wait()

        def wait_s(b):
            pltpu.make_async_copy(rows_v.at[b], out_hbm.at[pl.ds(0, _CH)],
                                  ssem.at[b]).wait()

        def start(i):
            b = i % D

            @pl.when(i < cpw)
            def _():
                pltpu.sync_copy(idx_hbm.at[pl.ds(base + i * _CH, _CH)],
                                idx_v[b])
                pltpu.async_copy(h_hbm.at[idx_v[b]], rows_v.at[b],
                                 gsem.at[b])

        for j in range(D - 1):
            start(j)
        for i in range(cpw0):
            b = i % D
            j = i + D - 1
            if j < cpw0:
                if j >= D:
                    @pl.when(j < cpw)
                    def _(j=j):
                        wait_s(j % D)
                start(j)

            @pl.when(i < cpw)
            def _(b=b, i=i):
                wait_g(b)
                pltpu.async_copy(rows_v.at[b],
                                 out_hbm.at[pl.ds(base + i * _CH, _CH)],
                                 ssem.at[b])
        for b in range(D):
            wait_s(b)

    return body


def _sc_gather(h, idx_pad, cpw0, cpw1):
    e_slice = _NS * (cpw0 + cpw1) * _CH
    mesh = plsc.VectorSubcoreMesh(core_axis_name="c", subcore_axis_name="s")
    k = pl.kernel(
        _make_gather_body(cpw0, cpw1),
        out_type=jax.ShapeDtypeStruct((e_slice, EF), jnp.float32),
        mesh=mesh,
        scratch_types=(
            [pltpu.VMEM((_CH,), jnp.int32)] * _DEPTH
            + [pltpu.VMEM((_DEPTH, _CH, EF), jnp.float32),
               pltpu.SemaphoreType.DMA((_DEPTH,)),
               pltpu.SemaphoreType.DMA((_DEPTH,))]
        ),
    )
    return k(h, idx_pad)


def _mlp_max_body(g_ref, xyz_ref, w1x_ref, b1_ref, w2t_ref, b2_ref, o_ref):
    xyzc = lax.dot_general(xyz_ref[0], w1x_ref[...],
                           dimension_numbers=(((0,), (0,)), ((), ())),
                           preferred_element_type=jnp.float32)
    pre = g_ref[...] + b1_ref[...] + xyzc
    act = jnp.where(pre >= 0, pre, 0.01 * pre)
    o2 = jnp.dot(act.astype(jnp.bfloat16), w2t_ref[...],
                 preferred_element_type=jnp.float32)
    bm = o2.shape[0]
    o_ref[...] = jnp.max(o2.reshape(bm // KNN, KNN, EF), axis=1) + b2_ref[...]


def _mlp_max(g, xyz_t, w1x_t, b1, w2_t, b2, n_nodes, bm):
    e = n_nodes * KNN
    grid = e // bm
    return pl.pallas_call(
        _mlp_max_body,
        grid=(grid,),
        in_specs=[
            pl.BlockSpec((bm, EF), lambda i: (i, 0)),
            pl.BlockSpec((1, 3, bm), lambda i: (i, 0, 0)),
            pl.BlockSpec((3, EF), lambda i: (0, 0)),
            pl.BlockSpec((1, EF), lambda i: (0, 0)),
            pl.BlockSpec((EF, EF), lambda i: (0, 0)),
            pl.BlockSpec((1, EF), lambda i: (0, 0)),
        ],
        out_specs=pl.BlockSpec((bm // KNN, EF), lambda i: (i, 0)),
        out_shape=jax.ShapeDtypeStruct((n_nodes, EF), jnp.float32),
    )(g, xyz_t, w1x_t, b1, w2_t, b2)


def kernel(input, KNN_idx, KNN_xyz, W1, b1, W2, b2):
    n = input.shape[0]
    e = KNN_idx.shape[0]

    idx = KNN_idx.astype(jnp.int32)
    e_pad = _NS * (_CPW0 + _CPW1) * _CH
    idx_pad = jnp.concatenate(
        [idx, jnp.zeros((e_pad - e,), dtype=jnp.int32)])

    w1f_t = W1[:, :EF].T
    w1x_t = W1[:, EF:].T
    w2_t = W2.T.astype(jnp.bfloat16)

    h = _h_matmul(input, w1f_t)
    g = _sc_gather(h, idx_pad, _CPW0, _CPW1)

    bm = 8000
    xyz_t = KNN_xyz.reshape(e // bm, bm, 3).transpose(0, 2, 1)

    return _mlp_max(g, xyz_t, w1x_t, b1.reshape(1, EF), w2_t,
                    b2.reshape(1, EF), n, bm)
